# baseline (device time: 15648 ns/iter reference)
import jax
import jax.numpy as jnp
from jax import lax
from jax.experimental import pallas as pl
from jax.experimental.pallas import tpu as pltpu

N_DEV = 4
B = 2
S_LOC = 128
S_GLOB = N_DEV * S_LOC
D = 512
HQ = 4
DH = 64
HD = HQ * DH
SCALE = 0.125
R_LOC = B * S_LOC
QSCALE = 56.0
INV_Q = 1.0 / QSCALE


def kernel(x, Wq, Wk, Wv, Wo):
    f32 = jnp.float32
    bf16 = jnp.bfloat16

    idx = jnp.arange(DH)
    evenrow = (idx % 2 == 0).astype(f32)[:, None]
    r64 = jnp.eye(DH, k=1) * evenrow - jnp.eye(DH, k=-1) * (1.0 - evenrow)
    r256 = jnp.kron(jnp.eye(HQ, dtype=f32), r64).astype(bf16)

    def body(x_ref, wq_ref, wk_ref, wv_ref, wo_ref, r_ref,
             out_ref, kv_slots, send_sems, recv_sems):
        my_pos = lax.axis_index("i")

        barrier_sem = pltpu.get_barrier_semaphore()
        for d in (2, 1, 3):
            pl.semaphore_signal(barrier_sem, inc=1,
                                device_id=((my_pos + d) % N_DEV,),
                                device_id_type=pl.DeviceIdType.MESH)

        xf = jnp.concatenate(
            [x_ref[0, :, :].astype(bf16), x_ref[1, :, :].astype(bf16)],
            axis=0)
        r16 = r_ref[:, :]

        seq = (lax.broadcasted_iota(jnp.int32, (S_LOC, DH), 0)
               + my_pos * S_LOC).astype(f32)
        j = lax.broadcasted_iota(jnp.int32, (S_LOC, DH), 1)
        e2 = (j - j % 2).astype(f32) * (1.0 / DH)
        ang = seq * jnp.exp(e2 * (-jnp.log(10000.0)))
        c64, s64 = jnp.cos(ang), jnp.sin(ang)
        c128 = jnp.concatenate([c64] * HQ, axis=1)
        s128 = jnp.concatenate([s64] * HQ, axis=1)
        cos_v = jnp.concatenate([c128] * B, axis=0)
        sin_v = jnp.concatenate([s128] * B, axis=0)

        k_pre = jnp.dot(xf, wk_ref[:, :].astype(bf16),
                        preferred_element_type=f32)
        v_pre = jnp.dot(xf, wv_ref[:, :].astype(bf16),
                        preferred_element_type=f32)
        k_rot = jnp.dot(k_pre.astype(bf16), r16,
                        preferred_element_type=f32)
        k16 = k_pre * cos_v + k_rot * sin_v
        kv_slots[0, :, 0:HD] = jnp.clip(
            jnp.round(k16 * QSCALE), -127.0, 127.0).astype(jnp.int8)
        kv_slots[0, :, HD:2 * HD] = jnp.clip(
            jnp.round(v_pre * QSCALE), -127.0, 127.0).astype(jnp.int8)

        pl.semaphore_wait(barrier_sem, N_DEV - 1)

        sends = []
        for d in (2, 1, 3):
            rdma = pltpu.make_async_remote_copy(
                src_ref=kv_slots.at[0],
                dst_ref=kv_slots.at[d],
                send_sem=send_sems.at[d],
                recv_sem=recv_sems.at[d],
                device_id=((my_pos + d) % N_DEV,),
                device_id_type=pl.DeviceIdType.MESH,
            )
            rdma.start()
            sends.append(rdma)

        q_pre = jnp.dot(xf, wq_ref[:, :].astype(bf16),
                        preferred_element_type=f32)
        q_rot = jnp.dot(q_pre.astype(bf16), r16,
                        preferred_element_type=f32)
        q16 = ((q_pre * cos_v + q_rot * sin_v) * SCALE).astype(bf16)

        s_blocks = [[None] * N_DEV for _ in range(B * HQ)]
        v_deqs = [None] * N_DEV
        for si, d in enumerate((0, 1, 3, 2)):
            if d != 0:
                recv = pltpu.make_async_remote_copy(
                    src_ref=kv_slots.at[d],
                    dst_ref=kv_slots.at[d],
                    send_sem=send_sems.at[d],
                    recv_sem=recv_sems.at[d],
                    device_id=(my_pos,),
                    device_id_type=pl.DeviceIdType.MESH,
                )
                recv.wait_recv()
            k_deq = (kv_slots[d, :, 0:HD].astype(f32) * INV_Q).astype(bf16)
            v_deqs[si] = (kv_slots[d, :, HD:2 * HD].astype(f32)
                          * INV_Q).astype(bf16)
            for b in range(B):
                k_d = k_deq[b * S_LOC:(b + 1) * S_LOC, :]
                for hh in range(HQ):
                    qh = q16[b * S_LOC:(b + 1) * S_LOC,
                             hh * DH:(hh + 1) * DH]
                    kh = k_d[:, hh * DH:(hh + 1) * DH]
                    s_blocks[b * HQ + hh][si] = lax.dot_general(
                        qh, kh, (((1,), (1,)), ((), ())),
                        preferred_element_type=f32)

        ctx_rows = []
        for b in range(B):
            ctx_parts = []
            for hh in range(HQ):
                s = jnp.concatenate(s_blocks[b * HQ + hh], axis=1)
                w = jnp.exp(s)
                r = 1.0 / jnp.sum(w, axis=1, keepdims=True)
                w16 = w.astype(bf16)
                ctx = jnp.zeros((S_LOC, DH), f32)
                for si in range(N_DEV):
                    v_d = v_deqs[si][b * S_LOC:(b + 1) * S_LOC,
                                     hh * DH:(hh + 1) * DH]
                    ctx += jnp.dot(w16[:, si * S_LOC:(si + 1) * S_LOC], v_d,
                                   preferred_element_type=f32)
                ctx_parts.append(ctx * r)
            ctx_rows.append(jnp.concatenate(ctx_parts, axis=1))
        ctx_full = jnp.concatenate(ctx_rows, axis=0).astype(bf16)
        out = jnp.dot(ctx_full, wo_ref[:, :].astype(bf16),
                      preferred_element_type=f32)
        for b in range(B):
            out_ref[b, :, :] = out[b * S_LOC:(b + 1) * S_LOC, :]

        for rdma in sends:
            rdma.wait_send()

    return pl.pallas_call(
        body,
        out_shape=jax.ShapeDtypeStruct((B, S_LOC, D), f32),
        in_specs=[pl.BlockSpec(memory_space=pltpu.VMEM)] * 6,
        out_specs=pl.BlockSpec(memory_space=pltpu.VMEM),
        scratch_shapes=[
            pltpu.VMEM((N_DEV, R_LOC, 2 * HD), jnp.int8),
            pltpu.SemaphoreType.DMA((N_DEV,)),
            pltpu.SemaphoreType.DMA((N_DEV,)),
        ],
        compiler_params=pltpu.CompilerParams(collective_id=0),
    )(x, Wq, Wk, Wv, Wo, r256)


# device time: 13327 ns/iter; 1.1742x vs baseline; 1.1742x over previous
import jax
import jax.numpy as jnp
from jax import lax
from jax.experimental import pallas as pl
from jax.experimental.pallas import tpu as pltpu

N_DEV = 4
B = 2
S_LOC = 128
S_GLOB = N_DEV * S_LOC
D = 512
HQ = 4
DH = 64
HD = HQ * DH
SCALE = 0.125
R_LOC = B * S_LOC
QSCALE = 56.0
INV_Q = 1.0 / QSCALE


def kernel(x, Wq, Wk, Wv, Wo):
    f32 = jnp.float32
    bf16 = jnp.bfloat16

    my = lax.axis_index("i")
    pos = (my * S_LOC + jnp.arange(S_LOC)).astype(f32)[:, None]
    inv = 1.0 / (10000.0 ** (jnp.arange(0, DH, 2).astype(f32) / DH))
    ang = pos * inv[None, :]
    cos = jnp.repeat(jnp.cos(ang), 2, axis=-1)
    sin = jnp.repeat(jnp.sin(ang), 2, axis=-1)
    cosb = jnp.tile(cos, (B, HQ)).astype(bf16)
    sinb = jnp.tile(sin, (B, HQ)).astype(bf16)

    idx = jnp.arange(DH)
    evenrow = (idx % 2 == 0).astype(f32)[:, None]
    r64 = jnp.eye(DH, k=1) * evenrow - jnp.eye(DH, k=-1) * (1.0 - evenrow)
    r256 = jnp.kron(jnp.eye(HQ, dtype=f32), r64).astype(bf16)

    def body(x_ref, wq_ref, wk_ref, wv_ref, wo_ref, r_ref, cos_ref, sin_ref,
             out_ref, kv_slots, send_sems, recv_sems):
        my_pos = lax.axis_index("i")

        barrier_sem = pltpu.get_barrier_semaphore()
        for d in (2, 1, 3):
            pl.semaphore_signal(barrier_sem, inc=1,
                                device_id=((my_pos + d) % N_DEV,),
                                device_id_type=pl.DeviceIdType.MESH)

        xf = jnp.concatenate(
            [x_ref[0, :, :].astype(bf16), x_ref[1, :, :].astype(bf16)],
            axis=0)
        r16 = r_ref[:, :]
        cos_v = cos_ref[:, :].astype(f32)
        sin_v = sin_ref[:, :].astype(f32)

        k_pre = jnp.dot(xf, wk_ref[:, :].astype(bf16),
                        preferred_element_type=f32)
        v_pre = jnp.dot(xf, wv_ref[:, :].astype(bf16),
                        preferred_element_type=f32)
        k_rot = jnp.dot(k_pre.astype(bf16), r16,
                        preferred_element_type=f32)
        k16 = k_pre * cos_v + k_rot * sin_v
        kv_slots[0, :, 0:HD] = jnp.clip(
            jnp.round(k16 * QSCALE), -127.0, 127.0).astype(jnp.int8)
        kv_slots[0, :, HD:2 * HD] = jnp.clip(
            jnp.round(v_pre * QSCALE), -127.0, 127.0).astype(jnp.int8)

        pl.semaphore_wait(barrier_sem, N_DEV - 1)

        sends = []
        for d in (2, 1, 3):
            rdma = pltpu.make_async_remote_copy(
                src_ref=kv_slots.at[0],
                dst_ref=kv_slots.at[d],
                send_sem=send_sems.at[d],
                recv_sem=recv_sems.at[d],
                device_id=((my_pos + d) % N_DEV,),
                device_id_type=pl.DeviceIdType.MESH,
            )
            rdma.start()
            sends.append(rdma)

        q_pre = jnp.dot(xf, wq_ref[:, :].astype(bf16),
                        preferred_element_type=f32)
        q_rot = jnp.dot(q_pre.astype(bf16), r16,
                        preferred_element_type=f32)
        q16 = ((q_pre * cos_v + q_rot * sin_v) * SCALE).astype(bf16)

        s_blocks = [[None] * N_DEV for _ in range(B * HQ)]
        v_deqs = [None] * N_DEV
        for si, d in enumerate((0, 1, 3, 2)):
            if d != 0:
                recv = pltpu.make_async_remote_copy(
                    src_ref=kv_slots.at[d],
                    dst_ref=kv_slots.at[d],
                    send_sem=send_sems.at[d],
                    recv_sem=recv_sems.at[d],
                    device_id=(my_pos,),
                    device_id_type=pl.DeviceIdType.MESH,
                )
                recv.wait_recv()
            k_deq = (kv_slots[d, :, 0:HD].astype(f32) * INV_Q).astype(bf16)
            v_deqs[si] = (kv_slots[d, :, HD:2 * HD].astype(f32)
                          * INV_Q).astype(bf16)
            for b in range(B):
                k_d = k_deq[b * S_LOC:(b + 1) * S_LOC, :]
                for hh in range(HQ):
                    qh = q16[b * S_LOC:(b + 1) * S_LOC,
                             hh * DH:(hh + 1) * DH]
                    kh = k_d[:, hh * DH:(hh + 1) * DH]
                    s_blocks[b * HQ + hh][si] = lax.dot_general(
                        qh, kh, (((1,), (1,)), ((), ())),
                        preferred_element_type=f32)

        ctx_rows = []
        for b in range(B):
            ctx_parts = []
            for hh in range(HQ):
                s = jnp.concatenate(s_blocks[b * HQ + hh], axis=1)
                w = jnp.exp(s)
                r = 1.0 / jnp.sum(w, axis=1, keepdims=True)
                w16 = w.astype(bf16)
                ctx = jnp.zeros((S_LOC, DH), f32)
                for si in range(N_DEV):
                    v_d = v_deqs[si][b * S_LOC:(b + 1) * S_LOC,
                                     hh * DH:(hh + 1) * DH]
                    ctx += jnp.dot(w16[:, si * S_LOC:(si + 1) * S_LOC], v_d,
                                   preferred_element_type=f32)
                ctx_parts.append(ctx * r)
            ctx_rows.append(jnp.concatenate(ctx_parts, axis=1))
        ctx_full = jnp.concatenate(ctx_rows, axis=0).astype(bf16)
        out = jnp.dot(ctx_full, wo_ref[:, :].astype(bf16),
                      preferred_element_type=f32)
        for b in range(B):
            out_ref[b, :, :] = out[b * S_LOC:(b + 1) * S_LOC, :]

        for rdma in sends:
            rdma.wait_send()

    return pl.pallas_call(
        body,
        out_shape=jax.ShapeDtypeStruct((B, S_LOC, D), f32),
        in_specs=[pl.BlockSpec(memory_space=pltpu.VMEM)] * 8,
        out_specs=pl.BlockSpec(memory_space=pltpu.VMEM),
        scratch_shapes=[
            pltpu.VMEM((N_DEV, R_LOC, 2 * HD), jnp.int8),
            pltpu.SemaphoreType.DMA((N_DEV,)),
            pltpu.SemaphoreType.DMA((N_DEV,)),
        ],
        compiler_params=pltpu.CompilerParams(collective_id=0),
    )(x, Wq, Wk, Wv, Wo, r256, cosb, sinb)
